# Initial kernel scaffold; baseline (speedup 1.0000x reference)
import jax
import jax.numpy as jnp
from jax.experimental import pallas as pl


def kernel(x, src_down, dst_down, src_up, dst_up, w_down, w_up):
    raise NotImplementedError("write your pallas kernel here")


# speedup vs baseline: 1.3029x; 1.3029x over previous
import jax
import jax.numpy as jnp
from jax import lax
from jax.experimental import pallas as pl
from jax.experimental.pallas import tpu as pltpu
from jax.experimental.pallas import tpu_sc as plsc

ND = 50000
NT = 10000
E = 150000
D = 512
L = 16
NS = 16
CW = 32
NQ = D // CW
QPC = NQ // 2
BE = 512
NB = 296
EP = NB * BE
KB = BE // 128
GB = 4
NG = NB // GB
WR = NB * BE // L
IPT = (NB + NS - 1) // NS
RMUL = D // CW
EPS = 1e-8
TRD = NT // NS
TRU = ND // NS

_f32 = jnp.float32
_i32 = jnp.int32


def _iota16():
    return lax.iota(_i32, L)


def _fill_zero_rows(ref, nrows, ncols):
    z = jnp.zeros((L,), _f32)

    def body(r, c):
        for j in range(ncols // L):
            ref[r, pl.ds(j * L, L)] = z
        return c

    lax.fori_loop(0, nrows, body, None)


def _seg_sums(dv, wv):
    iota = _iota16()
    sk, sw = plsc.sort_key_val(dv, wv)
    cs = plsc.cumsum(sw)
    excl = cs - sw
    skprev = sk.at[jnp.maximum(iota - 1, 0)].get(mode="promise_in_bounds")
    first = (iota == 0) | (sk != skprev)
    fidx = plsc.cummax(jnp.where(first, iota, 0))
    exa = excl.at[fidx].get(mode="promise_in_bounds")
    seg = cs - exa
    sknext = sk.at[jnp.minimum(iota + 1, L - 1)].get(
        mode="promise_in_bounds")
    last = (iota == L - 1) | (sk != sknext)
    return sk, seg, last


def _hist_block(didx4, wbuf4, k, table, lo, cap):
    for l in range(BE // L):
        dv = didx4[k, l // 8, pl.ds((l % 8) * L, L)]
        wv = wbuf4[k, pl.ds(l * L, L)]
        sk, seg, last = _seg_sums(dv, wv)
        inr = (sk >= lo) & (sk < lo + cap)
        lidx = jnp.clip(sk - lo, 0, cap - 1)
        plsc.addupdate_scatter(
            table, [lax.shift_right_logical(lidx, 4), lidx & 15], seg,
            mask=last & inr)


def _wnorm_block(didx4, wbuf4, k, table, lo, cap, cbuf):
    for l in range(BE // L):
        dv = didx4[k, l // 8, pl.ds((l % 8) * L, L)]
        wv = wbuf4[k, pl.ds(l * L, L)]
        inr = (dv >= lo) & (dv < lo + cap)
        lidx = jnp.clip(dv - lo, 0, cap - 1)
        nsv = plsc.load_gather(
            table, [lax.shift_right_logical(lidx, 4), lidx & 15])
        wn = wv / (nsv + EPS)
        cbuf[k * (BE // L) + l, pl.ds(0, L)] = jnp.where(inr, wn, 0.0)


def _scale_rows(rowbuf, wnbuf2):
    def body(i, _):
        for kk in range(4):
            e = i * 4 + kk
            wsp = plsc.load_gather(
                wnbuf2, [jnp.broadcast_to(e >> 4, (L,)),
                         jnp.broadcast_to(e & 15, (L,))])
            rowbuf[e, pl.ds(0, L)] = rowbuf[e, pl.ds(0, L)] * wsp
            rowbuf[e, pl.ds(L, L)] = rowbuf[e, pl.ds(L, L)] * wsp
        return _

    lax.fori_loop(0, BE // 4, body, None)


def _gather_scale_scatter(src_hbm, idx_mul, idx_add, sbuf, gidx, didx4,
                          wnbuf2, rowbuf, sem, acc_s):
    for l in range(BE // L):
        sv = sbuf[l // 8, pl.ds((l % 8) * L, L)]
        gidx[l // 8, pl.ds((l % 8) * L, L)] = sv * idx_mul + idx_add
    descs = []
    for j in range(KB):
        descs.append(pltpu.async_copy(
            src_hbm.at[gidx.at[j]], rowbuf.at[pl.ds(j * 128, 128)], sem))
    for dsc in descs:
        dsc.wait()
    _scale_rows(rowbuf, wnbuf2)
    for j in range(KB):
        pltpu.sync_copy(rowbuf.at[pl.ds(j * 128, 128)],
                        acc_s.at[didx4.at[0, j]], add=True)


def _down_body(x2, src4, dstd4, wd2, dstu4, wu2, coarse2, wun,
               sbuf, gidx, didx4, wbuf4, wnbuf2, rowbuf,
               tdn, tup, cbuf, pidx, zb16, sem,
               wnd_s, wun_s, acc_s):
    cid = lax.axis_index("c")
    sid = lax.axis_index("s")

    _fill_zero_rows(zb16, WR // NS, L)
    _fill_zero_rows(tdn, TRD // L + 1, L)
    _fill_zero_rows(tup, TRU // L + 1, L)
    pltpu.sync_copy(zb16, wnd_s.at[pl.ds((WR // NS) * sid, WR // NS)])
    pltpu.sync_copy(zb16, wun_s.at[pl.ds((WR // NS) * sid, WR // NS)])
    plsc.subcore_barrier()

    lo_d = sid * TRD
    lo_u = sid * TRU

    def p1_iter(g, _):
        pltpu.sync_copy(dstd4.at[pl.ds(g * GB, GB)], didx4)
        pltpu.sync_copy(wd2.at[pl.ds(g * GB, GB)], wbuf4)

        def blk(k, c):
            _hist_block(didx4, wbuf4, k, tdn, lo_d, TRD)
            return c

        lax.fori_loop(0, GB, blk, None)

        @pl.when(cid == 0)
        def _():
            pltpu.sync_copy(dstu4.at[pl.ds(g * GB, GB)], didx4)
            pltpu.sync_copy(wu2.at[pl.ds(g * GB, GB)], wbuf4)

            def blku(k, c):
                _hist_block(didx4, wbuf4, k, tup, lo_u, TRU)
                return c

            lax.fori_loop(0, GB, blku, None)

        return _

    lax.fori_loop(0, NG, p1_iter, None)
    plsc.subcore_barrier()

    def p2_iter(g, _):
        for l8 in range(8):
            pidx[0, pl.ds(l8 * L, L)] = _iota16() + l8 * L + g * 128
        _fill_zero_rows(cbuf, GB * BE // L, L)
        pltpu.sync_copy(dstd4.at[pl.ds(g * GB, GB)], didx4)
        pltpu.sync_copy(wd2.at[pl.ds(g * GB, GB)], wbuf4)

        def blk(k, c):
            _wnorm_block(didx4, wbuf4, k, tdn, lo_d, TRD, cbuf)
            return c

        lax.fori_loop(0, GB, blk, None)
        pltpu.sync_copy(cbuf, wnd_s.at[pidx.at[0]], add=True)

        @pl.when(cid == 0)
        def _():
            _fill_zero_rows(cbuf, GB * BE // L, L)
            pltpu.sync_copy(dstu4.at[pl.ds(g * GB, GB)], didx4)
            pltpu.sync_copy(wu2.at[pl.ds(g * GB, GB)], wbuf4)

            def blku(k, c):
                _wnorm_block(didx4, wbuf4, k, tup, lo_u, TRU, cbuf)
                return c

            lax.fori_loop(0, GB, blku, None)
            pltpu.sync_copy(cbuf, wun_s.at[pidx.at[0]], add=True)

        return _

    lax.fori_loop(0, NG, p2_iter, None)
    plsc.subcore_barrier()

    @pl.when(cid == 0)
    def _():
        pltpu.sync_copy(wun_s.at[pl.ds((WR // NS) * sid, WR // NS)],
                        wun.at[pl.ds((WR // NS) * sid, WR // NS)])

    def chunk(qq, _):
        q = cid * QPC + qq
        _fill_zero_rows(rowbuf, BE, CW)
        pltpu.sync_copy(rowbuf, acc_s.at[pl.ds(TRD * sid, BE)])
        pltpu.sync_copy(rowbuf.at[pl.ds(0, TRD - BE)],
                        acc_s.at[pl.ds(TRD * sid + BE, TRD - BE)])
        plsc.subcore_barrier()

        def p3_iter(i, c):
            b = sid + i * NS

            @pl.when(b < NB)
            def _():
                pltpu.sync_copy(src4.at[b], sbuf)
                pltpu.sync_copy(dstd4.at[pl.ds(b, 1)], didx4.at[pl.ds(0, 1)])
                pltpu.sync_copy(wnd_s.at[pl.ds(b * (BE // L), BE // L)],
                                wnbuf2)
                _gather_scale_scatter(x2, RMUL, q, sbuf, gidx, didx4,
                                      wnbuf2, rowbuf, sem, acc_s)

            return c

        lax.fori_loop(0, IPT, p3_iter, None)
        plsc.subcore_barrier()

        @pl.when(sid < 10)
        def _():
            pltpu.sync_copy(acc_s.at[pl.ds(1000 * sid, 1000)],
                            coarse2.at[pl.ds(q * NT + 1000 * sid, 1000)])

        plsc.subcore_barrier()
        return _

    lax.fori_loop(0, QPC, chunk, None)


def _up_body(coarse2, src4, dstu4, wun, out4,
             sbuf, gidx, didx4, wnbuf2, rowbuf, sem, acc_s):
    cid = lax.axis_index("c")
    sid = lax.axis_index("s")

    def chunk(qq, _):
        q = cid * QPC + qq
        _fill_zero_rows(rowbuf, BE, CW)
        for p in range(6):
            pltpu.sync_copy(rowbuf,
                            acc_s.at[pl.ds(TRU * sid + p * BE, BE)])
        pltpu.sync_copy(rowbuf.at[pl.ds(0, TRU - 6 * BE)],
                        acc_s.at[pl.ds(TRU * sid + 6 * BE, TRU - 6 * BE)])
        plsc.subcore_barrier()

        def p_iter(i, c):
            b = sid + i * NS

            @pl.when(b < NB)
            def _():
                pltpu.sync_copy(src4.at[b], sbuf)
                pltpu.sync_copy(dstu4.at[pl.ds(b, 1)], didx4.at[pl.ds(0, 1)])
                pltpu.sync_copy(wun.at[pl.ds(b * (BE // L), BE // L)],
                                wnbuf2)
                _gather_scale_scatter(coarse2, 1, q * NT, sbuf, gidx, didx4,
                                      wnbuf2, rowbuf, sem, acc_s)

            return c

        lax.fori_loop(0, IPT, p_iter, None)
        plsc.subcore_barrier()

        @pl.when(sid < 10)
        def _():
            pltpu.sync_copy(acc_s.at[pl.ds(5000 * sid, 5000)],
                            out4.at[pl.ds(q * ND + 5000 * sid, 5000)])

        plsc.subcore_barrier()
        return _

    lax.fori_loop(0, QPC, chunk, None)


_mesh = plsc.VectorSubcoreMesh(core_axis_name="c", subcore_axis_name="s")
_cparams = pltpu.CompilerParams(needs_layout_passes=False,
                                use_tc_tiling_on_sc=False)

_down = pl.kernel(
    _down_body,
    out_type=(jax.ShapeDtypeStruct((NQ * NT, CW), _f32),
              jax.ShapeDtypeStruct((WR, L), _f32)),
    mesh=_mesh,
    compiler_params=_cparams,
    scratch_types=(
        pltpu.VMEM((KB, 128), _i32),
        pltpu.VMEM((KB, 128), _i32),
        pltpu.VMEM((GB, KB, 128), _i32),
        pltpu.VMEM((GB, BE), _f32),
        pltpu.VMEM((BE // L, L), _f32),
        pltpu.VMEM((BE, CW), _f32),
        pltpu.VMEM((TRD // L + 1, L), _f32),
        pltpu.VMEM((TRU // L + 1, L), _f32),
        pltpu.VMEM((GB * BE // L, L), _f32),
        pltpu.VMEM((1, 128), _i32),
        pltpu.VMEM((WR // NS, L), _f32),
        pltpu.SemaphoreType.DMA,
        pltpu.VMEM_SHARED((WR, L), _f32),
        pltpu.VMEM_SHARED((WR, L), _f32),
        pltpu.VMEM_SHARED((NT, CW), _f32),
    ),
)

_up = pl.kernel(
    _up_body,
    out_type=jax.ShapeDtypeStruct((NQ * ND, CW), _f32),
    mesh=_mesh,
    compiler_params=_cparams,
    scratch_types=(
        pltpu.VMEM((KB, 128), _i32),
        pltpu.VMEM((KB, 128), _i32),
        pltpu.VMEM((GB, KB, 128), _i32),
        pltpu.VMEM((BE // L, L), _f32),
        pltpu.VMEM((BE, CW), _f32),
        pltpu.SemaphoreType.DMA,
        pltpu.VMEM_SHARED((ND, CW), _f32),
    ),
)


def kernel(x, src_down, dst_down, src_up, dst_up, w_down, w_up):
    x2 = x.reshape(2, ND * RMUL, CW)[1]
    pad = EP - E

    def pad1(a):
        return jnp.concatenate([a, jnp.zeros((pad,), a.dtype)])

    src_d4 = pad1(src_down).reshape(NB, KB, 128)
    dst_d4 = pad1(dst_down).reshape(NB, KB, 128)
    src_u4 = pad1(src_up).reshape(NB, KB, 128)
    dst_u4 = pad1(dst_up).reshape(NB, KB, 128)
    w_d2 = pad1(w_down).reshape(NB, BE)
    w_u2 = pad1(w_up).reshape(NB, BE)

    coarse2, wun = _down(x2, src_d4, dst_d4, w_d2, dst_u4, w_u2)
    out4 = _up(coarse2, src_u4, dst_u4, wun)
    out = out4.reshape(NQ, ND, CW).transpose(1, 0, 2)
    return out.reshape(1, 1, ND, D)


# speedup vs baseline: 1.3399x; 1.0284x over previous
import jax
import jax.numpy as jnp
from jax import lax
from jax.experimental import pallas as pl
from jax.experimental.pallas import tpu as pltpu
from jax.experimental.pallas import tpu_sc as plsc

ND = 50000
NT = 10000
E = 150000
D = 512
L = 16
NS = 16
CW = 32
NQ = D // CW
QPC = NQ // 2
BE = 256
NB = 592
EP = NB * BE
KB = BE // 128
VB = BE // L
GB = 8
NG = NB // GB
WR = NB * VB
IPT = NB // NS
RMUL = D // CW
EPS = 1e-8
TRD = NT // NS
TRU = 1568

_f32 = jnp.float32
_i32 = jnp.int32


def _iota16():
    return lax.iota(_i32, L)


def _fill_zero_rows(ref, nrows, ncols):
    z = jnp.zeros((L,), _f32)

    def body(r, c):
        for j in range(ncols // L):
            ref[r, pl.ds(j * L, L)] = z
        return c

    lax.fori_loop(0, nrows, body, None)


def _fill_zero_rows3(ref, p, nrows, ncols):
    z = jnp.zeros((L,), _f32)

    def body(r, c):
        for j in range(ncols // L):
            ref[p, r, pl.ds(j * L, L)] = z
        return c

    lax.fori_loop(0, nrows, body, None)


def _seg_sums(dv, wv):
    iota = _iota16()
    sk, sw = plsc.sort_key_val(dv, wv)
    cs = plsc.cumsum(sw)
    excl = cs - sw
    skprev = sk.at[jnp.maximum(iota - 1, 0)].get(mode="promise_in_bounds")
    first = (iota == 0) | (sk != skprev)
    fidx = plsc.cummax(jnp.where(first, iota, 0))
    exa = excl.at[fidx].get(mode="promise_in_bounds")
    seg = cs - exa
    sknext = sk.at[jnp.minimum(iota + 1, L - 1)].get(
        mode="promise_in_bounds")
    last = (iota == L - 1) | (sk != sknext)
    return sk, seg, last


def _hist_block(didx4, wbuf4, p, k, table, lo, cap):
    for l in range(VB):
        dv = didx4[p, k, l // 8, pl.ds((l % 8) * L, L)]
        wv = wbuf4[p, k, pl.ds(l * L, L)]
        sk, seg, last = _seg_sums(dv, wv)
        inr = (sk >= lo) & (sk < lo + cap)
        lidx = jnp.clip(sk - lo, 0, cap - 1)
        plsc.addupdate_scatter(
            table, [lax.shift_right_logical(lidx, 4), lidx & 15], seg,
            mask=last & inr)


def _wnorm_block(didx4, wbuf4, p, k, table, lo, cap, cbuf):
    for l in range(VB):
        dv = didx4[p, k, l // 8, pl.ds((l % 8) * L, L)]
        wv = wbuf4[p, k, pl.ds(l * L, L)]
        inr = (dv >= lo) & (dv < lo + cap)
        lidx = jnp.clip(dv - lo, 0, cap - 1)
        nsv = plsc.load_gather(
            table, [lax.shift_right_logical(lidx, 4), lidx & 15])
        wn = wv / (nsv + EPS)
        cbuf[k * VB + l, pl.ds(0, L)] = jnp.where(inr, wn, 0.0)


def _scale_rows(rowbuf, p, wnbuf2):
    def body(i, _):
        for kk in range(4):
            e = i * 4 + kk
            wsp = plsc.load_gather(
                wnbuf2, [jnp.broadcast_to(p, (L,)),
                         jnp.broadcast_to(e >> 4, (L,)),
                         jnp.broadcast_to(e & 15, (L,))])
            rowbuf[p, e, pl.ds(0, L)] = rowbuf[p, e, pl.ds(0, L)] * wsp
            rowbuf[p, e, pl.ds(L, L)] = rowbuf[p, e, pl.ds(L, L)] * wsp
        return _

    lax.fori_loop(0, BE // 4, body, None)


class _Pipe:
    def __init__(self, src_hbm, idx_mul, idx_add, stage_fn, sbuf, gidx,
                 didx, rowbuf, gsems, ssems, acc_s):
        self.src_hbm = src_hbm
        self.idx_mul = idx_mul
        self.idx_add = idx_add
        self.stage_fn = stage_fn
        self.sbuf = sbuf
        self.gidx = gidx
        self.didx = didx
        self.rowbuf = rowbuf
        self.gsems = gsems
        self.ssems = ssems
        self.acc_s = acc_s

    def setup(self, b, p):
        self.stage_fn(b, p)
        for l in range(KB * 8):
            sv = self.sbuf[p, l // 8, pl.ds((l % 8) * L, L)]
            self.gidx[p, l // 8, pl.ds((l % 8) * L, L)] = (
                sv * self.idx_mul + self.idx_add)
        for j in range(KB):
            pltpu.async_copy(
                self.src_hbm.at[self.gidx.at[p, j]],
                self.rowbuf.at[p, pl.ds(j * 128, 128)], self.gsems[p])

    def wait_scatter(self, p):
        for j in range(KB):
            pltpu.make_async_copy(
                self.rowbuf.at[p, pl.ds(j * 128, 128)],
                self.acc_s.at[self.didx.at[p, j]], self.ssems[p]).wait()

    def process(self, p, wnbuf2):
        for j in range(KB):
            pltpu.make_async_copy(
                self.src_hbm.at[self.gidx.at[p, j]],
                self.rowbuf.at[p, pl.ds(j * 128, 128)], self.gsems[p]).wait()
        _scale_rows(self.rowbuf, p, wnbuf2)
        for j in range(KB):
            pltpu.async_copy(
                self.rowbuf.at[p, pl.ds(j * 128, 128)],
                self.acc_s.at[self.didx.at[p, j]], self.ssems[p], add=True)

    def run(self, sid, wnbuf2):
        self.setup(sid, 0)

        def pair(ip, _):
            i0 = 2 * ip

            @pl.when(ip >= 1)
            def _():
                self.wait_scatter(1)

            self.setup(sid + (i0 + 1) * NS, 1)
            self.process(0, wnbuf2)
            self.wait_scatter(0)
            self.setup(sid + (i0 + 2) * NS, 0)
            self.process(1, wnbuf2)
            return _

        lax.fori_loop(0, (IPT - 1) // 2, pair, None)
        self.wait_scatter(1)
        self.process(0, wnbuf2)
        self.wait_scatter(0)


def _down_body(x2, src4, dstd4, wd2, dstu4, wu2, coarse2, wun2,
               sbuf, gidx, didx, wnbuf2, rowbuf,
               didx4, wbuf4, cbuf, pidx,
               tdn, tup, zb16,
               gsem0, gsem1, ssem0, ssem1,
               wnd_s, wun_s, acc_s):
    cid = lax.axis_index("c")
    sid = lax.axis_index("s")
    wid = cid * NS + sid

    _fill_zero_rows(zb16, WR // NS // 2, L)
    _fill_zero_rows(tdn, TRD // L + 1, L)
    _fill_zero_rows(tup, TRU // L, L)
    for h in range(2):
        pltpu.sync_copy(
            zb16, wnd_s.at[pl.ds((WR // NS) * sid + h * (WR // NS // 2),
                                 WR // NS // 2)])
        pltpu.sync_copy(
            zb16, wun_s.at[pl.ds((WR // NS) * sid + h * (WR // NS // 2),
                                 WR // NS // 2)])
    plsc.subcore_barrier()

    lo_d = sid * TRD
    lo_u = wid * TRU

    def p1_stage(g, p):
        pltpu.sync_copy(dstd4.at[pl.ds(g * GB, GB)], didx4.at[p, 0])
        pltpu.sync_copy(wd2.at[pl.ds(g * GB, GB)], wbuf4.at[p, 0])
        pltpu.sync_copy(dstu4.at[pl.ds(g * GB, GB)], didx4.at[p, 1])
        pltpu.sync_copy(wu2.at[pl.ds(g * GB, GB)], wbuf4.at[p, 1])

    def p1_iter(g, _):
        p1_stage(g, 0)

        def blk(k, c):
            _hist_block(didx4.at[0], wbuf4.at[0], 0, k, tdn, lo_d, TRD)
            _hist_block(didx4.at[0], wbuf4.at[0], 1, k, tup, lo_u, TRU)
            return c

        lax.fori_loop(0, GB, blk, None)
        return _

    lax.fori_loop(0, NG, p1_iter, None)
    plsc.subcore_barrier()

    def p2_iter(g, _):
        for l8 in range(8):
            pidx[0, pl.ds(l8 * L, L)] = _iota16() + l8 * L + g * 128
        p1_stage(g, 0)
        _fill_zero_rows3(cbuf, 0, GB * VB, L)
        _fill_zero_rows3(cbuf, 1, GB * VB, L)

        def blk(k, c):
            _wnorm_block(didx4.at[0], wbuf4.at[0], 0, k, tdn, lo_d, TRD,
                         cbuf.at[0])
            _wnorm_block(didx4.at[0], wbuf4.at[0], 1, k, tup, lo_u, TRU,
                         cbuf.at[1])
            return c

        lax.fori_loop(0, GB, blk, None)
        pltpu.sync_copy(cbuf.at[0], wnd_s.at[pidx.at[0]], add=True)
        pltpu.sync_copy(cbuf.at[1], wun_s.at[pidx.at[0]], add=True)
        return _

    lax.fori_loop(0, NG, p2_iter, None)
    plsc.subcore_barrier()

    pltpu.sync_copy(wun_s.at[pl.ds((WR // NS) * sid, WR // NS)],
                    wun2.at[cid, pl.ds((WR // NS) * sid, WR // NS)])

    def stage_fn(b, p):
        pltpu.sync_copy(src4.at[pl.ds(b, 1)], sbuf.at[pl.ds(p, 1)])
        pltpu.sync_copy(dstd4.at[pl.ds(b, 1)], didx.at[pl.ds(p, 1)])
        pltpu.sync_copy(wnd_s.at[pl.ds(b * VB, VB)], wnbuf2.at[p])

    def chunk(qq, _):
        q = cid * QPC + qq
        pipe = _Pipe(x2, RMUL, q, stage_fn, sbuf, gidx, didx, rowbuf,
                     (gsem0, gsem1), (ssem0, ssem1), acc_s)
        _fill_zero_rows3(rowbuf, 0, BE, CW)
        for s in range(2):
            pltpu.sync_copy(rowbuf.at[0],
                            acc_s.at[pl.ds(TRD * sid + s * BE, BE)])
        pltpu.sync_copy(rowbuf.at[0, pl.ds(0, TRD - 2 * BE)],
                        acc_s.at[pl.ds(TRD * sid + 2 * BE, TRD - 2 * BE)])
        plsc.subcore_barrier()
        pipe.run(sid, wnbuf2)
        plsc.subcore_barrier()

        @pl.when(sid < 10)
        def _():
            pltpu.sync_copy(acc_s.at[pl.ds(1000 * sid, 1000)],
                            coarse2.at[pl.ds(q * NT + 1000 * sid, 1000)])

        plsc.subcore_barrier()
        return _

    lax.fori_loop(0, QPC, chunk, None)


def _up_body(coarse2, src4, dstu4, wun2, out4,
             sbuf, gidx, didx, wnbuf2, wnbufb, rowbuf,
             gsem0, gsem1, ssem0, ssem1, acc_s):
    cid = lax.axis_index("c")
    sid = lax.axis_index("s")

    def stage_fn(b, p):
        pltpu.sync_copy(src4.at[pl.ds(b, 1)], sbuf.at[pl.ds(p, 1)])
        pltpu.sync_copy(dstu4.at[pl.ds(b, 1)], didx.at[pl.ds(p, 1)])
        pltpu.sync_copy(wun2.at[0, pl.ds(b * VB, VB)], wnbuf2.at[p])
        pltpu.sync_copy(wun2.at[1, pl.ds(b * VB, VB)], wnbufb.at[p])
        for r in range(VB):
            wnbuf2[p, r, pl.ds(0, L)] = (wnbuf2[p, r, pl.ds(0, L)] +
                                         wnbufb[p, r, pl.ds(0, L)])

    def chunk(qq, _):
        q = cid * QPC + qq
        pipe = _Pipe(coarse2, 1, q * NT, stage_fn, sbuf, gidx, didx, rowbuf,
                     (gsem0, gsem1), (ssem0, ssem1), acc_s)
        _fill_zero_rows3(rowbuf, 0, BE, CW)
        for s in range(12):
            pltpu.sync_copy(rowbuf.at[0],
                            acc_s.at[pl.ds(3125 * sid + s * BE, BE)])
        pltpu.sync_copy(rowbuf.at[0, pl.ds(0, 3125 - 12 * BE)],
                        acc_s.at[pl.ds(3125 * sid + 12 * BE, 3125 - 12 * BE)])
        plsc.subcore_barrier()
        pipe.run(sid, wnbuf2)
        plsc.subcore_barrier()

        @pl.when(sid < 10)
        def _():
            pltpu.sync_copy(acc_s.at[pl.ds(5000 * sid, 5000)],
                            out4.at[pl.ds(q * ND + 5000 * sid, 5000)])

        plsc.subcore_barrier()
        return _

    lax.fori_loop(0, QPC, chunk, None)


_mesh = plsc.VectorSubcoreMesh(core_axis_name="c", subcore_axis_name="s")
_cparams = pltpu.CompilerParams(needs_layout_passes=False,
                                use_tc_tiling_on_sc=False)

_down = pl.kernel(
    _down_body,
    out_type=(jax.ShapeDtypeStruct((NQ * NT, CW), _f32),
              jax.ShapeDtypeStruct((2, WR, L), _f32)),
    mesh=_mesh,
    compiler_params=_cparams,
    scratch_types=(
        pltpu.VMEM((2, KB, 128), _i32),
        pltpu.VMEM((2, KB, 128), _i32),
        pltpu.VMEM((2, KB, 128), _i32),
        pltpu.VMEM((2, VB, L), _f32),
        pltpu.VMEM((2, BE, CW), _f32),
        pltpu.VMEM((1, 2, GB, KB, 128), _i32),
        pltpu.VMEM((1, 2, GB, BE), _f32),
        pltpu.VMEM((2, GB * VB, L), _f32),
        pltpu.VMEM((1, 128), _i32),
        pltpu.VMEM((TRD // L + 1, L), _f32),
        pltpu.VMEM((TRU // L, L), _f32),
        pltpu.VMEM((WR // NS // 2, L), _f32),
        pltpu.SemaphoreType.DMA,
        pltpu.SemaphoreType.DMA,
        pltpu.SemaphoreType.DMA,
        pltpu.SemaphoreType.DMA,
        pltpu.VMEM_SHARED((WR, L), _f32),
        pltpu.VMEM_SHARED((WR, L), _f32),
        pltpu.VMEM_SHARED((NT, CW), _f32),
    ),
)

_up = pl.kernel(
    _up_body,
    out_type=jax.ShapeDtypeStruct((NQ * ND, CW), _f32),
    mesh=_mesh,
    compiler_params=_cparams,
    scratch_types=(
        pltpu.VMEM((2, KB, 128), _i32),
        pltpu.VMEM((2, KB, 128), _i32),
        pltpu.VMEM((2, KB, 128), _i32),
        pltpu.VMEM((2, VB, L), _f32),
        pltpu.VMEM((2, VB, L), _f32),
        pltpu.VMEM((2, BE, CW), _f32),
        pltpu.SemaphoreType.DMA,
        pltpu.SemaphoreType.DMA,
        pltpu.SemaphoreType.DMA,
        pltpu.SemaphoreType.DMA,
        pltpu.VMEM_SHARED((ND, CW), _f32),
    ),
)


def kernel(x, src_down, dst_down, src_up, dst_up, w_down, w_up):
    x2 = x.reshape(2, ND * RMUL, CW)[1]
    pad = EP - E

    def pad1(a):
        return jnp.concatenate([a, jnp.zeros((pad,), a.dtype)])

    src_d4 = pad1(src_down).reshape(NB, KB, 128)
    dst_d4 = pad1(dst_down).reshape(NB, KB, 128)
    src_u4 = pad1(src_up).reshape(NB, KB, 128)
    dst_u4 = pad1(dst_up).reshape(NB, KB, 128)
    w_d2 = pad1(w_down).reshape(NB, BE)
    w_u2 = pad1(w_up).reshape(NB, BE)

    coarse2, wun2 = _down(x2, src_d4, dst_d4, w_d2, dst_u4, w_u2)
    out4 = _up(coarse2, src_u4, dst_u4, wun2)
    out = out4.reshape(NQ, ND, CW).transpose(1, 0, 2)
    return out.reshape(1, 1, ND, D)


# speedup vs baseline: 2.4515x; 1.8297x over previous
import jax
import jax.numpy as jnp
from jax import lax
from jax.experimental import pallas as pl
from jax.experimental.pallas import tpu as pltpu
from jax.experimental.pallas import tpu_sc as plsc

ND = 50000
NT = 10000
E = 150000
D = 512
L = 16
NS = 16
CW = 32
NQ = D // CW
QPC = NQ // 2
BE = 256
NB = 592
EP = NB * BE
KB = BE // 128
VB = BE // L
GB = 8
NG = NB // GB
WR = NB * VB
IPT = NB // NS
SB = 8
RMUL = D // CW
EPS = 1e-8
TRD = NT // NS
TRU = 1568

_f32 = jnp.float32
_i32 = jnp.int32


def _iota16():
    return lax.iota(_i32, L)


def _fill_zero_rows(ref, nrows, ncols):
    z = jnp.zeros((L,), _f32)

    def body(r, c):
        for j in range(ncols // L):
            ref[r, pl.ds(j * L, L)] = z
        return c

    lax.fori_loop(0, nrows, body, None)


def _fill_zero_rows3(ref, p, nrows, ncols):
    z = jnp.zeros((L,), _f32)

    def body(r, c):
        for j in range(ncols // L):
            ref[p, r, pl.ds(j * L, L)] = z
        return c

    lax.fori_loop(0, nrows, body, None)


def _hist_block(didx4, wbuf4, d, k, table, lo, cap):
    for l in range(VB):
        dv = didx4[d, k, l // 8, pl.ds((l % 8) * L, L)]
        wv = wbuf4[d, k, pl.ds(l * L, L)]
        inr = (dv >= lo) & (dv < lo + cap)
        lidx = jnp.clip(dv - lo, 0, cap - 1)
        plsc.addupdate_scatter(
            table, [lax.shift_right_logical(lidx, 4), lidx & 15], wv,
            mask=inr)


def _invert_table(table, nrows):
    def body(r, c):
        v = table[r, pl.ds(0, L)]
        table[r, pl.ds(0, L)] = 1.0 / (v + EPS)
        return c

    lax.fori_loop(0, nrows, body, None)


def _wnorm_block(didx4, wbuf4, d, k, table, lo, cap, cbuf):
    for l in range(VB):
        dv = didx4[d, k, l // 8, pl.ds((l % 8) * L, L)]
        wv = wbuf4[d, k, pl.ds(l * L, L)]
        inr = (dv >= lo) & (dv < lo + cap)
        lidx = jnp.clip(dv - lo, 0, cap - 1)
        rsv = plsc.load_gather(
            table, [lax.shift_right_logical(lidx, 4), lidx & 15])
        cbuf[k * VB + l, pl.ds(0, L)] = jnp.where(inr, wv * rsv, 0.0)


def _scale_rows(rowbuf, p, wn8, sp, k):
    def body(i, _):
        for kk in range(4):
            e = i * 4 + kk
            wsp = plsc.load_gather(
                wn8, [jnp.broadcast_to(sp, (L,)),
                      jnp.broadcast_to(k * L + (e >> 4), (L,)),
                      jnp.broadcast_to(e & 15, (L,))])
            rowbuf[p, e, pl.ds(0, L)] = rowbuf[p, e, pl.ds(0, L)] * wsp
            rowbuf[p, e, pl.ds(L, L)] = rowbuf[p, e, pl.ds(L, L)] * wsp
        return _

    lax.fori_loop(0, BE // 4, body, None)


def _proj_blocks(sid, src_hbm, idx_mul, idx_add, stage_fn,
                 gidx8, didx8, wn8, rowbuf, gsems, ssems, acc_s):
    def fire_gather(p, sp, k):
        for j in range(KB):
            pltpu.async_copy(
                src_hbm.at[gidx8.at[sp, k, j]],
                rowbuf.at[p, pl.ds(j * 128, 128)], gsems[p])

    def wait_scatter(p, sp, k):
        for j in range(KB):
            pltpu.make_async_copy(
                rowbuf.at[p, pl.ds(j * 128, 128)],
                acc_s.at[didx8.at[sp, k, j]], ssems[p]).wait()

    def process(p, sp, k):
        for j in range(KB):
            pltpu.make_async_copy(
                src_hbm.at[gidx8.at[sp, k, j]],
                rowbuf.at[p, pl.ds(j * 128, 128)], gsems[p]).wait()
        _scale_rows(rowbuf, p, wn8, sp, k)
        for j in range(KB):
            pltpu.async_copy(
                rowbuf.at[p, pl.ds(j * 128, 128)],
                acc_s.at[didx8.at[sp, k, j]], ssems[p], add=True)

    def super_body(ss, _):
        stage_fn(sid, ss, 0, SB, idx_mul, idx_add)
        fire_gather(0, 0, 0)
        for n in range(1, SB):
            fire_gather(n & 1, 0, n)
            process((n - 1) & 1, 0, n - 1)
            wait_scatter((n - 1) & 1, 0, n - 1)
        process((SB - 1) & 1, 0, SB - 1)
        wait_scatter((SB - 1) & 1, 0, SB - 1)
        return _

    nfull = IPT // SB
    lax.fori_loop(0, nfull, super_body, None)
    ntail = IPT - nfull * SB
    stage_fn(sid, nfull, 0, ntail, idx_mul, idx_add)
    fire_gather(0, 0, 0)
    for n in range(1, ntail):
        fire_gather(n & 1, 0, n)
        process((n - 1) & 1, 0, n - 1)
        wait_scatter((n - 1) & 1, 0, n - 1)
    process((ntail - 1) & 1, 0, ntail - 1)
    wait_scatter((ntail - 1) & 1, 0, ntail - 1)


def _mk_stage(src4, dst4, gidx8, didx8, wn8, wn_rows_fn):
    def stage_fn(sid, ss, sp, sbn, idx_mul, idx_add):
        b0 = sid * IPT + ss * SB
        pltpu.sync_copy(src4.at[pl.ds(b0, sbn)],
                        gidx8.at[sp, pl.ds(0, sbn)])
        pltpu.sync_copy(dst4.at[pl.ds(b0, sbn)],
                        didx8.at[sp, pl.ds(0, sbn)])
        wn_rows_fn(b0, sp, sbn)
        for k in range(sbn):
            for l in range(KB * 8):
                sv = gidx8[sp, k, l // 8, pl.ds((l % 8) * L, L)]
                gidx8[sp, k, l // 8, pl.ds((l % 8) * L, L)] = (
                    sv * idx_mul + idx_add)

    return stage_fn


def _down_body(x2, src4, dstd4, wd2, dstu4, wu2, coarse2, wun2,
               gidx8, didx8, wn8, rowbuf,
               didx4, wbuf4, cbuf, pidx,
               tdn, tup, zb16,
               gsem0, gsem1, ssem0, ssem1, stsem,
               wnd_s, wun_s, acc_s):
    cid = lax.axis_index("c")
    sid = lax.axis_index("s")
    wid = cid * NS + sid

    _fill_zero_rows(zb16, WR // NS // 2, L)
    _fill_zero_rows(tdn, TRD // L + 1, L)
    _fill_zero_rows(tup, TRU // L, L)
    for h in range(2):
        pltpu.sync_copy(
            zb16, wnd_s.at[pl.ds((WR // NS) * sid + h * (WR // NS // 2),
                                 WR // NS // 2)])
        pltpu.sync_copy(
            zb16, wun_s.at[pl.ds((WR // NS) * sid + h * (WR // NS // 2),
                                 WR // NS // 2)])
    plsc.subcore_barrier()

    lo_d = sid * TRD
    lo_u = wid * TRU

    def stage_grp(g, p):
        pltpu.async_copy(dstd4.at[pl.ds(g * GB, GB)], didx4.at[p, 0], stsem)
        pltpu.async_copy(wd2.at[pl.ds(g * GB, GB)], wbuf4.at[p, 0], stsem)
        pltpu.async_copy(dstu4.at[pl.ds(g * GB, GB)], didx4.at[p, 1], stsem)
        pltpu.async_copy(wu2.at[pl.ds(g * GB, GB)], wbuf4.at[p, 1], stsem)

    def wait_grp(g, p):
        pltpu.make_async_copy(
            dstd4.at[pl.ds(g * GB, GB)], didx4.at[p, 0], stsem).wait()
        pltpu.make_async_copy(
            wd2.at[pl.ds(g * GB, GB)], wbuf4.at[p, 0], stsem).wait()
        pltpu.make_async_copy(
            dstu4.at[pl.ds(g * GB, GB)], didx4.at[p, 1], stsem).wait()
        pltpu.make_async_copy(
            wu2.at[pl.ds(g * GB, GB)], wbuf4.at[p, 1], stsem).wait()

    def p1_grp(g, p):
        def blk(k, c):
            _hist_block(didx4.at[p], wbuf4.at[p], 0, k, tdn, lo_d, TRD)
            _hist_block(didx4.at[p], wbuf4.at[p], 1, k, tup, lo_u, TRU)
            return c

        lax.fori_loop(0, GB, blk, None)

    stage_grp(0, 0)

    def p1_pair(ip, _):
        g0 = 2 * ip
        wait_grp(g0, 0)
        stage_grp(g0 + 1, 1)
        p1_grp(g0, 0)
        wait_grp(g0 + 1, 1)

        @pl.when(ip < NG // 2 - 1)
        def _():
            stage_grp(g0 + 2, 0)

        p1_grp(g0 + 1, 1)
        return _

    lax.fori_loop(0, NG // 2, p1_pair, None)
    _invert_table(tdn, TRD // L + 1)
    _invert_table(tup, TRU // L)
    plsc.subcore_barrier()

    def p2_grp(g, p):
        for l8 in range(8):
            pidx[0, pl.ds(l8 * L, L)] = _iota16() + l8 * L + g * 128
        _fill_zero_rows3(cbuf, 0, GB * VB, L)
        _fill_zero_rows3(cbuf, 1, GB * VB, L)

        def blk(k, c):
            _wnorm_block(didx4.at[p], wbuf4.at[p], 0, k, tdn, lo_d, TRD,
                         cbuf.at[0])
            _wnorm_block(didx4.at[p], wbuf4.at[p], 1, k, tup, lo_u, TRU,
                         cbuf.at[1])
            return c

        lax.fori_loop(0, GB, blk, None)
        pltpu.sync_copy(cbuf.at[0], wnd_s.at[pidx.at[0]], add=True)
        pltpu.sync_copy(cbuf.at[1], wun_s.at[pidx.at[0]], add=True)

    stage_grp(0, 0)

    def p2_pair(ip, _):
        g0 = 2 * ip
        wait_grp(g0, 0)
        stage_grp(g0 + 1, 1)
        p2_grp(g0, 0)
        wait_grp(g0 + 1, 1)

        @pl.when(ip < NG // 2 - 1)
        def _():
            stage_grp(g0 + 2, 0)

        p2_grp(g0 + 1, 1)
        return _

    lax.fori_loop(0, NG // 2, p2_pair, None)
    plsc.subcore_barrier()

    pltpu.sync_copy(wun_s.at[pl.ds((WR // NS) * sid, WR // NS)],
                    wun2.at[cid, pl.ds((WR // NS) * sid, WR // NS)])

    def wn_rows(b0, sp, sbn):
        pltpu.sync_copy(wnd_s.at[pl.ds(b0 * VB, sbn * VB)],
                        wn8.at[sp, pl.ds(0, sbn * VB)])

    stage_fn = _mk_stage(src4, dstd4, gidx8, didx8, wn8, wn_rows)

    def chunk(qq, _):
        q = cid * QPC + qq
        _fill_zero_rows3(rowbuf, 0, BE, CW)
        for s in range(2):
            pltpu.sync_copy(rowbuf.at[0],
                            acc_s.at[pl.ds(TRD * sid + s * BE, BE)])
        pltpu.sync_copy(rowbuf.at[0, pl.ds(0, TRD - 2 * BE)],
                        acc_s.at[pl.ds(TRD * sid + 2 * BE, TRD - 2 * BE)])
        plsc.subcore_barrier()
        _proj_blocks(sid, x2, RMUL, q, stage_fn, gidx8, didx8, wn8,
                     rowbuf, (gsem0, gsem1), (ssem0, ssem1), acc_s)
        plsc.subcore_barrier()

        @pl.when(sid < 10)
        def _():
            pltpu.sync_copy(acc_s.at[pl.ds(1000 * sid, 1000)],
                            coarse2.at[pl.ds(q * NT + 1000 * sid, 1000)])

        plsc.subcore_barrier()
        return _

    lax.fori_loop(0, QPC, chunk, None)


def _up_body(coarse2, src4, dstu4, wun2, out4,
             gidx8, didx8, wn8, wn8b, rowbuf,
             gsem0, gsem1, ssem0, ssem1, acc_s):
    cid = lax.axis_index("c")
    sid = lax.axis_index("s")

    def wn_rows(b0, sp, sbn):
        pltpu.sync_copy(wun2.at[0, pl.ds(b0 * VB, sbn * VB)],
                        wn8.at[sp, pl.ds(0, sbn * VB)])
        pltpu.sync_copy(wun2.at[1, pl.ds(b0 * VB, sbn * VB)],
                        wn8b.at[pl.ds(0, sbn * VB)])
        for r in range(sbn * VB):
            wn8[sp, r, pl.ds(0, L)] = (wn8[sp, r, pl.ds(0, L)] +
                                       wn8b[r, pl.ds(0, L)])

    stage_fn = _mk_stage(src4, dstu4, gidx8, didx8, wn8, wn_rows)

    def chunk(qq, _):
        q = cid * QPC + qq
        _fill_zero_rows3(rowbuf, 0, BE, CW)
        for s in range(12):
            pltpu.sync_copy(rowbuf.at[0],
                            acc_s.at[pl.ds(3125 * sid + s * BE, BE)])
        pltpu.sync_copy(rowbuf.at[0, pl.ds(0, 3125 - 12 * BE)],
                        acc_s.at[pl.ds(3125 * sid + 12 * BE, 3125 - 12 * BE)])
        plsc.subcore_barrier()
        _proj_blocks(sid, coarse2, 1, q * NT, stage_fn, gidx8, didx8, wn8,
                     rowbuf, (gsem0, gsem1), (ssem0, ssem1), acc_s)
        plsc.subcore_barrier()

        @pl.when(sid < 10)
        def _():
            pltpu.sync_copy(acc_s.at[pl.ds(5000 * sid, 5000)],
                            out4.at[pl.ds(q * ND + 5000 * sid, 5000)])

        plsc.subcore_barrier()
        return _

    lax.fori_loop(0, QPC, chunk, None)


_mesh = plsc.VectorSubcoreMesh(core_axis_name="c", subcore_axis_name="s")
_cparams = pltpu.CompilerParams(needs_layout_passes=False,
                                use_tc_tiling_on_sc=False)

_down = pl.kernel(
    _down_body,
    out_type=(jax.ShapeDtypeStruct((NQ * NT, CW), _f32),
              jax.ShapeDtypeStruct((2, WR, L), _f32)),
    mesh=_mesh,
    compiler_params=_cparams,
    scratch_types=(
        pltpu.VMEM((1, SB, KB, 128), _i32),
        pltpu.VMEM((1, SB, KB, 128), _i32),
        pltpu.VMEM((1, SB * VB, L), _f32),
        pltpu.VMEM((2, BE, CW), _f32),
        pltpu.VMEM((2, 2, GB, KB, 128), _i32),
        pltpu.VMEM((2, 2, GB, BE), _f32),
        pltpu.VMEM((2, GB * VB, L), _f32),
        pltpu.VMEM((1, 128), _i32),
        pltpu.VMEM((TRD // L + 1, L), _f32),
        pltpu.VMEM((TRU // L, L), _f32),
        pltpu.VMEM((WR // NS // 2, L), _f32),
        pltpu.SemaphoreType.DMA,
        pltpu.SemaphoreType.DMA,
        pltpu.SemaphoreType.DMA,
        pltpu.SemaphoreType.DMA,
        pltpu.SemaphoreType.DMA,
        pltpu.VMEM_SHARED((WR, L), _f32),
        pltpu.VMEM_SHARED((WR, L), _f32),
        pltpu.VMEM_SHARED((NT, CW), _f32),
    ),
)

_up = pl.kernel(
    _up_body,
    out_type=jax.ShapeDtypeStruct((NQ * ND, CW), _f32),
    mesh=_mesh,
    compiler_params=_cparams,
    scratch_types=(
        pltpu.VMEM((1, SB, KB, 128), _i32),
        pltpu.VMEM((1, SB, KB, 128), _i32),
        pltpu.VMEM((1, SB * VB, L), _f32),
        pltpu.VMEM((SB * VB, L), _f32),
        pltpu.VMEM((2, BE, CW), _f32),
        pltpu.SemaphoreType.DMA,
        pltpu.SemaphoreType.DMA,
        pltpu.SemaphoreType.DMA,
        pltpu.SemaphoreType.DMA,
        pltpu.VMEM_SHARED((ND, CW), _f32),
    ),
)


def kernel(x, src_down, dst_down, src_up, dst_up, w_down, w_up):
    x2 = x[0, 1, 0].reshape(ND * RMUL, CW)
    pad = EP - E

    def pad1(a):
        return jnp.concatenate([a, jnp.zeros((pad,), a.dtype)])

    src_d4 = pad1(src_down).reshape(NB, KB, 128)
    dst_d4 = pad1(dst_down).reshape(NB, KB, 128)
    src_u4 = pad1(src_up).reshape(NB, KB, 128)
    dst_u4 = pad1(dst_up).reshape(NB, KB, 128)
    w_d2 = pad1(w_down).reshape(NB, BE)
    w_u2 = pad1(w_up).reshape(NB, BE)

    coarse2, wun2 = _down(x2, src_d4, dst_d4, w_d2, dst_u4, w_u2)
    out4 = _up(coarse2, src_u4, dst_u4, wun2)
    out = out4.reshape(NQ, ND, CW).transpose(1, 0, 2)
    return out.reshape(1, 1, ND, D)


# speedup vs baseline: 2.7699x; 1.1299x over previous
import jax
import jax.numpy as jnp
from jax import lax
from jax.experimental import pallas as pl
from jax.experimental.pallas import tpu as pltpu
from jax.experimental.pallas import tpu_sc as plsc

ND = 50000
NT = 10000
E = 150000
D = 512
L = 16
NS = 16
CW = 32
NQ = D // CW
QPC = NQ // 2
BE = 256
NB = 592
EP = NB * BE
KB = BE // 128
VB = BE // L
GB = 8
NG = NB // GB
WR = NB * VB
IPT = NB // NS
SB = 8
RMUL = D // CW
EPS = 1e-8
TRD = NT // NS
TRU = 1568

_f32 = jnp.float32
_i32 = jnp.int32


def _iota16():
    return lax.iota(_i32, L)


def _fill_zero_rows(ref, nrows, ncols):
    z = jnp.zeros((L,), _f32)

    def body(r, c):
        for j in range(ncols // L):
            ref[r, pl.ds(j * L, L)] = z
        return c

    lax.fori_loop(0, nrows, body, None)


def _fill_zero_rows3(ref, p, nrows, ncols):
    z = jnp.zeros((L,), _f32)

    def body(r, c):
        for j in range(ncols // L):
            ref[p, r, pl.ds(j * L, L)] = z
        return c

    lax.fori_loop(0, nrows, body, None)


def _hist_block(didx4, wbuf4, d, k, table, lo, cap):
    for l in range(VB):
        dv = didx4[d, k, l // 8, pl.ds((l % 8) * L, L)]
        wv = wbuf4[d, k, pl.ds(l * L, L)]
        inr = (dv >= lo) & (dv < lo + cap)
        lidx = jnp.clip(dv - lo, 0, cap - 1)
        plsc.addupdate_scatter(
            table, [lax.shift_right_logical(lidx, 4), lidx & 15], wv,
            mask=inr)


def _invert_table(table, nrows):
    def body(r, c):
        v = table[r, pl.ds(0, L)]
        table[r, pl.ds(0, L)] = 1.0 / (v + EPS)
        return c

    lax.fori_loop(0, nrows, body, None)


def _wnorm_block(didx4, wbuf4, d, k, table, lo, cap, cbuf):
    for l in range(VB):
        dv = didx4[d, k, l // 8, pl.ds((l % 8) * L, L)]
        wv = wbuf4[d, k, pl.ds(l * L, L)]
        inr = (dv >= lo) & (dv < lo + cap)
        lidx = jnp.clip(dv - lo, 0, cap - 1)
        rsv = plsc.load_gather(
            table, [lax.shift_right_logical(lidx, 4), lidx & 15])
        cbuf[k * VB + l, pl.ds(0, L)] = jnp.where(inr, wv * rsv, 0.0)


def _scale_rows(rowbuf, p, wn8, sp, k):
    def body(i, _):
        for kk in range(8):
            e = i * 8 + kk
            wsp = plsc.load_gather(
                wn8, [jnp.broadcast_to(sp, (L,)),
                      jnp.broadcast_to(k * L + (e >> 4), (L,)),
                      jnp.broadcast_to(e & 15, (L,))])
            rowbuf[p, e, pl.ds(0, L)] = rowbuf[p, e, pl.ds(0, L)] * wsp
            rowbuf[p, e, pl.ds(L, L)] = rowbuf[p, e, pl.ds(L, L)] * wsp
        return _

    lax.fori_loop(0, BE // 8, body, None)


def _proj_blocks(sid, src_hbm, idx_mul, idx_add, stage_fn,
                 gidx8, didx8, wn8, rowbuf, gsems, ssems, acc_s):
    def fire_gather(p, sp, k):
        for j in range(KB):
            pltpu.async_copy(
                src_hbm.at[gidx8.at[sp, k, j]],
                rowbuf.at[p, pl.ds(j * 128, 128)], gsems[p])

    def wait_scatter(p, sp, k):
        for j in range(KB):
            pltpu.make_async_copy(
                rowbuf.at[p, pl.ds(j * 128, 128)],
                acc_s.at[didx8.at[sp, k, j]], ssems[p]).wait()

    def process(p, sp, k):
        for j in range(KB):
            pltpu.make_async_copy(
                src_hbm.at[gidx8.at[sp, k, j]],
                rowbuf.at[p, pl.ds(j * 128, 128)], gsems[p]).wait()
        _scale_rows(rowbuf, p, wn8, sp, k)
        for j in range(KB):
            pltpu.async_copy(
                rowbuf.at[p, pl.ds(j * 128, 128)],
                acc_s.at[didx8.at[sp, k, j]], ssems[p], add=True)

    def run_blocks(nblk):
        fire_gather(0, 0, 0)
        for n in range(1, nblk):
            if n >= 2:
                wait_scatter(n & 1, 0, n - 2)
            fire_gather(n & 1, 0, n)
            process((n - 1) & 1, 0, n - 1)
        process((nblk - 1) & 1, 0, nblk - 1)
        wait_scatter((nblk - 2) & 1, 0, nblk - 2)
        wait_scatter((nblk - 1) & 1, 0, nblk - 1)

    def super_body(ss, _):
        stage_fn(sid, ss, 0, SB, idx_mul, idx_add)
        run_blocks(SB)
        return _

    nfull = IPT // SB
    lax.fori_loop(0, nfull, super_body, None)
    ntail = IPT - nfull * SB
    stage_fn(sid, nfull, 0, ntail, idx_mul, idx_add)
    run_blocks(ntail)


def _mk_stage(src4, dst4, gidx8, didx8, wn8, wn_rows_fn):
    def stage_fn(sid, ss, sp, sbn, idx_mul, idx_add):
        b0 = sid * IPT + ss * SB
        pltpu.sync_copy(src4.at[pl.ds(b0, sbn)],
                        gidx8.at[sp, pl.ds(0, sbn)])
        pltpu.sync_copy(dst4.at[pl.ds(b0, sbn)],
                        didx8.at[sp, pl.ds(0, sbn)])
        wn_rows_fn(b0, sp, sbn)
        for k in range(sbn):
            for l in range(KB * 8):
                sv = gidx8[sp, k, l // 8, pl.ds((l % 8) * L, L)]
                gidx8[sp, k, l // 8, pl.ds((l % 8) * L, L)] = (
                    sv * idx_mul + idx_add)

    return stage_fn


def _down_body(x2, src4, dstd4, wd2, dstu4, wu2, coarse2, wun2,
               gidx8, didx8, wn8, rowbuf,
               didx4, wbuf4, cbuf, pidx,
               tdn, tup, zb16,
               gsem0, gsem1, ssem0, ssem1, stsem, csem,
               wnd_s, wun_s, acc_s):
    cid = lax.axis_index("c")
    sid = lax.axis_index("s")
    wid = cid * NS + sid

    _fill_zero_rows(zb16, WR // NS // 2, L)
    _fill_zero_rows(tdn, TRD // L + 1, L)
    _fill_zero_rows(tup, TRU // L, L)
    for h in range(2):
        pltpu.sync_copy(
            zb16, wnd_s.at[pl.ds((WR // NS) * sid + h * (WR // NS // 2),
                                 WR // NS // 2)])
        pltpu.sync_copy(
            zb16, wun_s.at[pl.ds((WR // NS) * sid + h * (WR // NS // 2),
                                 WR // NS // 2)])
    plsc.subcore_barrier()

    lo_d = sid * TRD
    lo_u = wid * TRU

    def stage_grp(g, p):
        pltpu.async_copy(dstd4.at[pl.ds(g * GB, GB)], didx4.at[p, 0], stsem)
        pltpu.async_copy(wd2.at[pl.ds(g * GB, GB)], wbuf4.at[p, 0], stsem)
        pltpu.async_copy(dstu4.at[pl.ds(g * GB, GB)], didx4.at[p, 1], stsem)
        pltpu.async_copy(wu2.at[pl.ds(g * GB, GB)], wbuf4.at[p, 1], stsem)

    def wait_grp(g, p):
        pltpu.make_async_copy(
            dstd4.at[pl.ds(g * GB, GB)], didx4.at[p, 0], stsem).wait()
        pltpu.make_async_copy(
            wd2.at[pl.ds(g * GB, GB)], wbuf4.at[p, 0], stsem).wait()
        pltpu.make_async_copy(
            dstu4.at[pl.ds(g * GB, GB)], didx4.at[p, 1], stsem).wait()
        pltpu.make_async_copy(
            wu2.at[pl.ds(g * GB, GB)], wbuf4.at[p, 1], stsem).wait()

    def p1_grp(g, p):
        def blk(k, c):
            _hist_block(didx4.at[p], wbuf4.at[p], 0, k, tdn, lo_d, TRD)
            _hist_block(didx4.at[p], wbuf4.at[p], 1, k, tup, lo_u, TRU)
            return c

        lax.fori_loop(0, GB, blk, None)

    stage_grp(0, 0)

    def p1_pair(ip, _):
        g0 = 2 * ip
        wait_grp(g0, 0)
        stage_grp(g0 + 1, 1)
        p1_grp(g0, 0)
        wait_grp(g0 + 1, 1)

        @pl.when(ip < NG // 2 - 1)
        def _():
            stage_grp(g0 + 2, 0)

        p1_grp(g0 + 1, 1)
        return _

    lax.fori_loop(0, NG // 2, p1_pair, None)
    _invert_table(tdn, TRD // L + 1)
    _invert_table(tup, TRU // L)
    plsc.subcore_barrier()

    def wait_p2_scatter(p):
        pltpu.make_async_copy(cbuf.at[p, 0], wnd_s.at[pidx.at[p]],
                              csem).wait()
        pltpu.make_async_copy(cbuf.at[p, 1], wun_s.at[pidx.at[p]],
                              csem).wait()

    def p2_grp(g, p, first):
        if not first:
            wait_p2_scatter(p)
        for l8 in range(8):
            pidx[p, pl.ds(l8 * L, L)] = _iota16() + l8 * L + g * 128
        _fill_zero_rows3(cbuf.at[p], 0, GB * VB, L)
        _fill_zero_rows3(cbuf.at[p], 1, GB * VB, L)

        def blk(k, c):
            _wnorm_block(didx4.at[p], wbuf4.at[p], 0, k, tdn, lo_d, TRD,
                         cbuf.at[p, 0])
            _wnorm_block(didx4.at[p], wbuf4.at[p], 1, k, tup, lo_u, TRU,
                         cbuf.at[p, 1])
            return c

        lax.fori_loop(0, GB, blk, None)
        pltpu.async_copy(cbuf.at[p, 0], wnd_s.at[pidx.at[p]], csem,
                         add=True)
        pltpu.async_copy(cbuf.at[p, 1], wun_s.at[pidx.at[p]], csem,
                         add=True)

    stage_grp(0, 0)

    def p2_pair(ip, _):
        g0 = 2 * ip
        wait_grp(g0, 0)
        stage_grp(g0 + 1, 1)
        p2_grp(g0, 0, False)
        wait_grp(g0 + 1, 1)

        @pl.when(ip < NG // 2 - 1)
        def _():
            stage_grp(g0 + 2, 0)

        p2_grp(g0 + 1, 1, False)
        return _

    wait_grp(0, 0)
    stage_grp(1, 1)
    p2_grp(0, 0, True)
    wait_grp(1, 1)
    stage_grp(2, 0)
    p2_grp(1, 1, True)
    lax.fori_loop(1, NG // 2, p2_pair, None)
    wait_p2_scatter(0)
    wait_p2_scatter(1)
    plsc.subcore_barrier()

    pltpu.sync_copy(wun_s.at[pl.ds((WR // NS) * sid, WR // NS)],
                    wun2.at[cid, pl.ds((WR // NS) * sid, WR // NS)])

    def wn_rows(b0, sp, sbn):
        pltpu.sync_copy(wnd_s.at[pl.ds(b0 * VB, sbn * VB)],
                        wn8.at[sp, pl.ds(0, sbn * VB)])

    stage_fn = _mk_stage(src4, dstd4, gidx8, didx8, wn8, wn_rows)

    def chunk(qq, _):
        q = cid * QPC + qq
        _fill_zero_rows3(rowbuf, 0, BE, CW)
        for s in range(2):
            pltpu.sync_copy(rowbuf.at[0],
                            acc_s.at[pl.ds(TRD * sid + s * BE, BE)])
        pltpu.sync_copy(rowbuf.at[0, pl.ds(0, TRD - 2 * BE)],
                        acc_s.at[pl.ds(TRD * sid + 2 * BE, TRD - 2 * BE)])
        plsc.subcore_barrier()
        _proj_blocks(sid, x2, RMUL, q, stage_fn, gidx8, didx8, wn8,
                     rowbuf, (gsem0, gsem1), (ssem0, ssem1), acc_s)
        plsc.subcore_barrier()

        @pl.when(sid < 10)
        def _():
            pltpu.sync_copy(acc_s.at[pl.ds(1000 * sid, 1000)],
                            coarse2.at[pl.ds(q * NT + 1000 * sid, 1000)])

        plsc.subcore_barrier()
        return _

    lax.fori_loop(0, QPC, chunk, None)


def _up_body(coarse2, src4, dstu4, wun2, out4,
             gidx8, didx8, wn8, wn8b, rowbuf,
             gsem0, gsem1, ssem0, ssem1, acc_s):
    cid = lax.axis_index("c")
    sid = lax.axis_index("s")

    def wn_rows(b0, sp, sbn):
        pltpu.sync_copy(wun2.at[0, pl.ds(b0 * VB, sbn * VB)],
                        wn8.at[sp, pl.ds(0, sbn * VB)])
        pltpu.sync_copy(wun2.at[1, pl.ds(b0 * VB, sbn * VB)],
                        wn8b.at[pl.ds(0, sbn * VB)])
        for r in range(sbn * VB):
            wn8[sp, r, pl.ds(0, L)] = (wn8[sp, r, pl.ds(0, L)] +
                                       wn8b[r, pl.ds(0, L)])

    stage_fn = _mk_stage(src4, dstu4, gidx8, didx8, wn8, wn_rows)

    def chunk(qq, _):
        q = cid * QPC + qq
        _fill_zero_rows3(rowbuf, 0, BE, CW)
        for s in range(12):
            pltpu.sync_copy(rowbuf.at[0],
                            acc_s.at[pl.ds(3125 * sid + s * BE, BE)])
        pltpu.sync_copy(rowbuf.at[0, pl.ds(0, 3125 - 12 * BE)],
                        acc_s.at[pl.ds(3125 * sid + 12 * BE, 3125 - 12 * BE)])
        plsc.subcore_barrier()
        _proj_blocks(sid, coarse2, 1, q * NT, stage_fn, gidx8, didx8, wn8,
                     rowbuf, (gsem0, gsem1), (ssem0, ssem1), acc_s)
        plsc.subcore_barrier()

        @pl.when(sid < 10)
        def _():
            pltpu.sync_copy(
                acc_s.at[pl.ds(5000 * sid, 5000)],
                out4.at[pl.ds(5000 * sid, 5000), pl.ds(q * CW, CW)])

        plsc.subcore_barrier()
        return _

    lax.fori_loop(0, QPC, chunk, None)


_mesh = plsc.VectorSubcoreMesh(core_axis_name="c", subcore_axis_name="s")
_cparams = pltpu.CompilerParams(needs_layout_passes=False,
                                use_tc_tiling_on_sc=False)

_down = pl.kernel(
    _down_body,
    out_type=(jax.ShapeDtypeStruct((NQ * NT, CW), _f32),
              jax.ShapeDtypeStruct((2, WR, L), _f32)),
    mesh=_mesh,
    compiler_params=_cparams,
    scratch_types=(
        pltpu.VMEM((1, SB, KB, 128), _i32),
        pltpu.VMEM((1, SB, KB, 128), _i32),
        pltpu.VMEM((1, SB * VB, L), _f32),
        pltpu.VMEM((2, BE, CW), _f32),
        pltpu.VMEM((2, 2, GB, KB, 128), _i32),
        pltpu.VMEM((2, 2, GB, BE), _f32),
        pltpu.VMEM((2, 2, GB * VB, L), _f32),
        pltpu.VMEM((2, 128), _i32),
        pltpu.VMEM((TRD // L + 1, L), _f32),
        pltpu.VMEM((TRU // L, L), _f32),
        pltpu.VMEM((WR // NS // 2, L), _f32),
        pltpu.SemaphoreType.DMA,
        pltpu.SemaphoreType.DMA,
        pltpu.SemaphoreType.DMA,
        pltpu.SemaphoreType.DMA,
        pltpu.SemaphoreType.DMA,
        pltpu.SemaphoreType.DMA,
        pltpu.VMEM_SHARED((WR, L), _f32),
        pltpu.VMEM_SHARED((WR, L), _f32),
        pltpu.VMEM_SHARED((NT, CW), _f32),
    ),
)

_up = pl.kernel(
    _up_body,
    out_type=jax.ShapeDtypeStruct((ND, D), _f32),
    mesh=_mesh,
    compiler_params=_cparams,
    scratch_types=(
        pltpu.VMEM((1, SB, KB, 128), _i32),
        pltpu.VMEM((1, SB, KB, 128), _i32),
        pltpu.VMEM((1, SB * VB, L), _f32),
        pltpu.VMEM((SB * VB, L), _f32),
        pltpu.VMEM((2, BE, CW), _f32),
        pltpu.SemaphoreType.DMA,
        pltpu.SemaphoreType.DMA,
        pltpu.SemaphoreType.DMA,
        pltpu.SemaphoreType.DMA,
        pltpu.VMEM_SHARED((ND, CW), _f32),
    ),
)


def kernel(x, src_down, dst_down, src_up, dst_up, w_down, w_up):
    x2 = x[0, 1, 0].reshape(ND * RMUL, CW)
    pad = EP - E

    def pad1(a):
        return jnp.concatenate([a, jnp.zeros((pad,), a.dtype)])

    src_d4 = pad1(src_down).reshape(NB, KB, 128)
    dst_d4 = pad1(dst_down).reshape(NB, KB, 128)
    src_u4 = pad1(src_up).reshape(NB, KB, 128)
    dst_u4 = pad1(dst_up).reshape(NB, KB, 128)
    w_d2 = pad1(w_down).reshape(NB, BE)
    w_u2 = pad1(w_up).reshape(NB, BE)

    coarse2, wun2 = _down(x2, src_d4, dst_d4, w_d2, dst_u4, w_u2)
    out4 = _up(coarse2, src_u4, dst_u4, wun2)
    return out4.reshape(1, 1, ND, D)


# speedup vs baseline: 3.0353x; 1.0958x over previous
import jax
import jax.numpy as jnp
from jax import lax
from jax.experimental import pallas as pl
from jax.experimental.pallas import tpu as pltpu
from jax.experimental.pallas import tpu_sc as plsc

ND = 50000
NT = 10000
E = 150000
D = 512
L = 16
NS = 16
CW = 32
NQ = D // CW
QPC = NQ // 2
BE = 256
NB = 592
EP = NB * BE
KB = BE // 128
VB = BE // L
GB = 8
NG = NB // GB
WR = NB * VB
IPT = NB // NS
SB = 8
RMUL = D // CW
EPS = 1e-8
TRD = NT // NS
TRU = 1568

_f32 = jnp.float32
_i32 = jnp.int32


def _iota16():
    return lax.iota(_i32, L)


def _fill_zero_rows(ref, nrows, ncols):
    z = jnp.zeros((L,), _f32)

    def body(r, c):
        for j in range(ncols // L):
            ref[r, pl.ds(j * L, L)] = z
        return c

    lax.fori_loop(0, nrows, body, None)


def _fill_zero_rows3(ref, p, nrows, ncols):
    z = jnp.zeros((L,), _f32)

    def body(r, c):
        for j in range(ncols // L):
            ref[p, r, pl.ds(j * L, L)] = z
        return c

    lax.fori_loop(0, nrows, body, None)


def _hist_block(didx4, wbuf4, d, k, table, lo, cap):
    for l in range(VB):
        dv = didx4[d, k, l // 8, pl.ds((l % 8) * L, L)]
        wv = wbuf4[d, k, pl.ds(l * L, L)]
        inr = (dv >= lo) & (dv < lo + cap)
        lidx = jnp.clip(dv - lo, 0, cap - 1)
        plsc.addupdate_scatter(
            table, [lax.shift_right_logical(lidx, 4), lidx & 15], wv,
            mask=inr)


def _invert_table(table, nrows):
    def body(r, c):
        v = table[r, pl.ds(0, L)]
        table[r, pl.ds(0, L)] = 1.0 / (v + EPS)
        return c

    lax.fori_loop(0, nrows, body, None)


def _wnorm_block(didx4, wbuf4, d, k, table, lo, cap, cbuf):
    for l in range(VB):
        dv = didx4[d, k, l // 8, pl.ds((l % 8) * L, L)]
        wv = wbuf4[d, k, pl.ds(l * L, L)]
        inr = (dv >= lo) & (dv < lo + cap)
        lidx = jnp.clip(dv - lo, 0, cap - 1)
        rsv = plsc.load_gather(
            table, [lax.shift_right_logical(lidx, 4), lidx & 15])
        cbuf[k * VB + l, pl.ds(0, L)] = jnp.where(inr, wv * rsv, 0.0)


def _scale_rows(rowbuf, p, wn8, sp, k, nsub):
    def body(i, _):
        for kk in range(8):
            e = i * 8 + kk
            wsp = plsc.load_gather(
                wn8, [jnp.broadcast_to(sp, (L,)),
                      jnp.broadcast_to(k * L + (e >> 4), (L,)),
                      jnp.broadcast_to(e & 15, (L,))])
            for sv in range(nsub):
                rowbuf[p, e, pl.ds(sv * L, L)] = (
                    rowbuf[p, e, pl.ds(sv * L, L)] * wsp)
        return _

    lax.fori_loop(0, BE // 8, body, None)


def _proj_blocks(sid, src_hbm, idx_mul, idx_add, stage_fn,
                 gidx8, didx8, wn8, rowbuf, gsems, ssems, acc_s, nsub):
    def fire_gather(p, sp, k):
        for j in range(KB):
            pltpu.async_copy(
                src_hbm.at[gidx8.at[sp, k, j]],
                rowbuf.at[p, pl.ds(j * 128, 128)], gsems[p])

    def wait_scatter(p, sp, k):
        for j in range(KB):
            pltpu.make_async_copy(
                rowbuf.at[p, pl.ds(j * 128, 128)],
                acc_s.at[didx8.at[sp, k, j]], ssems[p]).wait()

    def process(p, sp, k):
        for j in range(KB):
            pltpu.make_async_copy(
                src_hbm.at[gidx8.at[sp, k, j]],
                rowbuf.at[p, pl.ds(j * 128, 128)], gsems[p]).wait()
        _scale_rows(rowbuf, p, wn8, sp, k, nsub)
        for j in range(KB):
            pltpu.async_copy(
                rowbuf.at[p, pl.ds(j * 128, 128)],
                acc_s.at[didx8.at[sp, k, j]], ssems[p], add=True)

    def run_blocks(nblk):
        fire_gather(0, 0, 0)
        for n in range(1, nblk):
            if n >= 2:
                wait_scatter(n & 1, 0, n - 2)
            fire_gather(n & 1, 0, n)
            process((n - 1) & 1, 0, n - 1)
        process((nblk - 1) & 1, 0, nblk - 1)
        wait_scatter((nblk - 2) & 1, 0, nblk - 2)
        wait_scatter((nblk - 1) & 1, 0, nblk - 1)

    def super_body(ss, _):
        stage_fn(sid, ss, 0, SB, idx_mul, idx_add)
        run_blocks(SB)
        return _

    nfull = IPT // SB
    lax.fori_loop(0, nfull, super_body, None)
    ntail = IPT - nfull * SB
    stage_fn(sid, nfull, 0, ntail, idx_mul, idx_add)
    run_blocks(ntail)


def _mk_stage(src4, dst4, gidx8, didx8, wn8, wn_rows_fn):
    def stage_fn(sid, ss, sp, sbn, idx_mul, idx_add):
        b0 = sid * IPT + ss * SB
        pltpu.sync_copy(src4.at[pl.ds(b0, sbn)],
                        gidx8.at[sp, pl.ds(0, sbn)])
        pltpu.sync_copy(dst4.at[pl.ds(b0, sbn)],
                        didx8.at[sp, pl.ds(0, sbn)])
        wn_rows_fn(b0, sp, sbn)
        for k in range(sbn):
            for l in range(KB * 8):
                sv = gidx8[sp, k, l // 8, pl.ds((l % 8) * L, L)]
                gidx8[sp, k, l // 8, pl.ds((l % 8) * L, L)] = (
                    sv * idx_mul + idx_add)

    return stage_fn


def _down_body(x2, src4, dstd4, wd2, dstu4, wu2, coarse2, wun2,
               gidx8, didx8, wn8, rowbuf,
               didx4, wbuf4, cbuf, pidx,
               tdn, tup, zb16,
               gsem0, gsem1, ssem0, ssem1, stsem, csem,
               wnd_s, wun_s, acc_s):
    cid = lax.axis_index("c")
    sid = lax.axis_index("s")
    wid = cid * NS + sid

    _fill_zero_rows(zb16, WR // NS // 2, L)
    _fill_zero_rows(tdn, TRD // L + 1, L)
    _fill_zero_rows(tup, TRU // L, L)
    for h in range(2):
        pltpu.sync_copy(
            zb16, wnd_s.at[pl.ds((WR // NS) * sid + h * (WR // NS // 2),
                                 WR // NS // 2)])
        pltpu.sync_copy(
            zb16, wun_s.at[pl.ds((WR // NS) * sid + h * (WR // NS // 2),
                                 WR // NS // 2)])
    plsc.subcore_barrier()

    lo_d = sid * TRD
    lo_u = wid * TRU

    def stage_grp(g, p):
        pltpu.async_copy(dstd4.at[pl.ds(g * GB, GB)], didx4.at[p, 0], stsem)
        pltpu.async_copy(wd2.at[pl.ds(g * GB, GB)], wbuf4.at[p, 0], stsem)
        pltpu.async_copy(dstu4.at[pl.ds(g * GB, GB)], didx4.at[p, 1], stsem)
        pltpu.async_copy(wu2.at[pl.ds(g * GB, GB)], wbuf4.at[p, 1], stsem)

    def wait_grp(g, p):
        pltpu.make_async_copy(
            dstd4.at[pl.ds(g * GB, GB)], didx4.at[p, 0], stsem).wait()
        pltpu.make_async_copy(
            wd2.at[pl.ds(g * GB, GB)], wbuf4.at[p, 0], stsem).wait()
        pltpu.make_async_copy(
            dstu4.at[pl.ds(g * GB, GB)], didx4.at[p, 1], stsem).wait()
        pltpu.make_async_copy(
            wu2.at[pl.ds(g * GB, GB)], wbuf4.at[p, 1], stsem).wait()

    def p1_grp(g, p):
        def blk(k, c):
            _hist_block(didx4.at[p], wbuf4.at[p], 0, k, tdn, lo_d, TRD)
            _hist_block(didx4.at[p], wbuf4.at[p], 1, k, tup, lo_u, TRU)
            return c

        lax.fori_loop(0, GB, blk, None)

    stage_grp(0, 0)

    def p1_pair(ip, _):
        g0 = 2 * ip
        wait_grp(g0, 0)
        stage_grp(g0 + 1, 1)
        p1_grp(g0, 0)
        wait_grp(g0 + 1, 1)

        @pl.when(ip < NG // 2 - 1)
        def _():
            stage_grp(g0 + 2, 0)

        p1_grp(g0 + 1, 1)
        return _

    lax.fori_loop(0, NG // 2, p1_pair, None)
    _invert_table(tdn, TRD // L + 1)
    _invert_table(tup, TRU // L)
    plsc.subcore_barrier()

    def wait_p2_scatter(p):
        pltpu.make_async_copy(cbuf.at[p, 0], wnd_s.at[pidx.at[p]],
                              csem).wait()
        pltpu.make_async_copy(cbuf.at[p, 1], wun_s.at[pidx.at[p]],
                              csem).wait()

    def p2_grp(g, p, first):
        if not first:
            wait_p2_scatter(p)
        for l8 in range(8):
            pidx[p, pl.ds(l8 * L, L)] = _iota16() + l8 * L + g * 128
        _fill_zero_rows3(cbuf.at[p], 0, GB * VB, L)
        _fill_zero_rows3(cbuf.at[p], 1, GB * VB, L)

        def blk(k, c):
            _wnorm_block(didx4.at[p], wbuf4.at[p], 0, k, tdn, lo_d, TRD,
                         cbuf.at[p, 0])
            _wnorm_block(didx4.at[p], wbuf4.at[p], 1, k, tup, lo_u, TRU,
                         cbuf.at[p, 1])
            return c

        lax.fori_loop(0, GB, blk, None)
        pltpu.async_copy(cbuf.at[p, 0], wnd_s.at[pidx.at[p]], csem,
                         add=True)
        pltpu.async_copy(cbuf.at[p, 1], wun_s.at[pidx.at[p]], csem,
                         add=True)

    stage_grp(0, 0)

    def p2_pair(ip, _):
        g0 = 2 * ip
        wait_grp(g0, 0)
        stage_grp(g0 + 1, 1)
        p2_grp(g0, 0, False)
        wait_grp(g0 + 1, 1)

        @pl.when(ip < NG // 2 - 1)
        def _():
            stage_grp(g0 + 2, 0)

        p2_grp(g0 + 1, 1, False)
        return _

    wait_grp(0, 0)
    stage_grp(1, 1)
    p2_grp(0, 0, True)
    wait_grp(1, 1)
    stage_grp(2, 0)
    p2_grp(1, 1, True)
    lax.fori_loop(1, NG // 2, p2_pair, None)
    wait_p2_scatter(0)
    wait_p2_scatter(1)
    plsc.subcore_barrier()

    pltpu.sync_copy(wun_s.at[pl.ds((WR // NS) * sid, WR // NS)],
                    wun2.at[cid, pl.ds((WR // NS) * sid, WR // NS)])

    def wn_rows(b0, sp, sbn):
        pltpu.sync_copy(wnd_s.at[pl.ds(b0 * VB, sbn * VB)],
                        wn8.at[sp, pl.ds(0, sbn * VB)])

    stage_fn = _mk_stage(src4, dstd4, gidx8, didx8, wn8, wn_rows)

    def chunk(qq, _):
        q = cid * (QPC // 2) + qq
        _fill_zero_rows3(rowbuf, 0, BE, 2 * CW)
        for s in range(2):
            pltpu.sync_copy(rowbuf.at[0],
                            acc_s.at[pl.ds(TRD * sid + s * BE, BE)])
        pltpu.sync_copy(rowbuf.at[0, pl.ds(0, TRD - 2 * BE)],
                        acc_s.at[pl.ds(TRD * sid + 2 * BE, TRD - 2 * BE)])
        plsc.subcore_barrier()
        _proj_blocks(sid, x2, RMUL // 2, q, stage_fn, gidx8, didx8, wn8,
                     rowbuf, (gsem0, gsem1), (ssem0, ssem1), acc_s, 4)
        plsc.subcore_barrier()

        @pl.when(sid < 10)
        def _():
            pltpu.sync_copy(acc_s.at[pl.ds(1000 * sid, 1000)],
                            coarse2.at[pl.ds(q * NT + 1000 * sid, 1000)])

        plsc.subcore_barrier()
        return _

    lax.fori_loop(0, QPC // 2, chunk, None)


def _up_body(coarse2, src4, dstu4, wun2, out4,
             gidx8, didx8, wn8, wn8b, rowbuf,
             gsem0, gsem1, ssem0, ssem1, acc_s):
    cid = lax.axis_index("c")
    sid = lax.axis_index("s")

    def wn_rows(b0, sp, sbn):
        pltpu.sync_copy(wun2.at[0, pl.ds(b0 * VB, sbn * VB)],
                        wn8.at[sp, pl.ds(0, sbn * VB)])
        pltpu.sync_copy(wun2.at[1, pl.ds(b0 * VB, sbn * VB)],
                        wn8b.at[pl.ds(0, sbn * VB)])
        for r in range(sbn * VB):
            wn8[sp, r, pl.ds(0, L)] = (wn8[sp, r, pl.ds(0, L)] +
                                       wn8b[r, pl.ds(0, L)])

    stage_fn = _mk_stage(src4, dstu4, gidx8, didx8, wn8, wn_rows)

    def chunk(qq, _):
        q = cid * QPC + qq
        _fill_zero_rows3(rowbuf, 0, BE, CW)
        for s in range(12):
            pltpu.sync_copy(rowbuf.at[0],
                            acc_s.at[pl.ds(3125 * sid + s * BE, BE)])
        pltpu.sync_copy(rowbuf.at[0, pl.ds(0, 3125 - 12 * BE)],
                        acc_s.at[pl.ds(3125 * sid + 12 * BE, 3125 - 12 * BE)])
        plsc.subcore_barrier()
        _proj_blocks(sid, coarse2, 2,
                     2 * NT * lax.shift_right_logical(q, 1) + (q & 1),
                     stage_fn, gidx8, didx8, wn8,
                     rowbuf, (gsem0, gsem1), (ssem0, ssem1), acc_s, 2)
        plsc.subcore_barrier()

        @pl.when(sid < 10)
        def _():
            pltpu.sync_copy(
                acc_s.at[pl.ds(5000 * sid, 5000)],
                out4.at[pl.ds(5000 * sid, 5000), pl.ds(q * CW, CW)])

        plsc.subcore_barrier()
        return _

    lax.fori_loop(0, QPC, chunk, None)


_mesh = plsc.VectorSubcoreMesh(core_axis_name="c", subcore_axis_name="s")
_cparams = pltpu.CompilerParams(needs_layout_passes=False,
                                use_tc_tiling_on_sc=False)

_down = pl.kernel(
    _down_body,
    out_type=(jax.ShapeDtypeStruct((NQ * NT // 2, 2 * CW), _f32),
              jax.ShapeDtypeStruct((2, WR, L), _f32)),
    mesh=_mesh,
    compiler_params=_cparams,
    scratch_types=(
        pltpu.VMEM((1, SB, KB, 128), _i32),
        pltpu.VMEM((1, SB, KB, 128), _i32),
        pltpu.VMEM((1, SB * VB, L), _f32),
        pltpu.VMEM((2, BE, 2 * CW), _f32),
        pltpu.VMEM((2, 2, GB, KB, 128), _i32),
        pltpu.VMEM((2, 2, GB, BE), _f32),
        pltpu.VMEM((2, 2, GB * VB, L), _f32),
        pltpu.VMEM((2, 128), _i32),
        pltpu.VMEM((TRD // L + 1, L), _f32),
        pltpu.VMEM((TRU // L, L), _f32),
        pltpu.VMEM((WR // NS // 2, L), _f32),
        pltpu.SemaphoreType.DMA,
        pltpu.SemaphoreType.DMA,
        pltpu.SemaphoreType.DMA,
        pltpu.SemaphoreType.DMA,
        pltpu.SemaphoreType.DMA,
        pltpu.SemaphoreType.DMA,
        pltpu.VMEM_SHARED((WR, L), _f32),
        pltpu.VMEM_SHARED((WR, L), _f32),
        pltpu.VMEM_SHARED((NT, 2 * CW), _f32),
    ),
)

_up = pl.kernel(
    _up_body,
    out_type=jax.ShapeDtypeStruct((ND, D), _f32),
    mesh=_mesh,
    compiler_params=_cparams,
    scratch_types=(
        pltpu.VMEM((1, SB, KB, 128), _i32),
        pltpu.VMEM((1, SB, KB, 128), _i32),
        pltpu.VMEM((1, SB * VB, L), _f32),
        pltpu.VMEM((SB * VB, L), _f32),
        pltpu.VMEM((2, BE, CW), _f32),
        pltpu.SemaphoreType.DMA,
        pltpu.SemaphoreType.DMA,
        pltpu.SemaphoreType.DMA,
        pltpu.SemaphoreType.DMA,
        pltpu.VMEM_SHARED((ND, CW), _f32),
    ),
)


def kernel(x, src_down, dst_down, src_up, dst_up, w_down, w_up):
    x2 = x[0, 1, 0].reshape(ND * RMUL // 2, 2 * CW)
    pad = EP - E

    def pad1(a):
        return jnp.concatenate([a, jnp.zeros((pad,), a.dtype)])

    src_d4 = pad1(src_down).reshape(NB, KB, 128)
    dst_d4 = pad1(dst_down).reshape(NB, KB, 128)
    src_u4 = pad1(src_up).reshape(NB, KB, 128)
    dst_u4 = pad1(dst_up).reshape(NB, KB, 128)
    w_d2 = pad1(w_down).reshape(NB, BE)
    w_u2 = pad1(w_up).reshape(NB, BE)

    coarse2, wun2 = _down(x2, src_d4, dst_d4, w_d2, dst_u4, w_u2)
    out4 = _up(coarse2.reshape(NQ * NT, CW), src_u4, dst_u4, wun2)
    return out4.reshape(1, 1, ND, D)
